# baseline (speedup 1.0000x reference)
import functools

import jax
import jax.numpy as jnp
from jax import lax
from jax.experimental import pallas as pl
from jax.experimental.pallas import tpu as pltpu
from jax.experimental.pallas import tpu_sc as plsc

B, F, D, V, DIN = 4096, 26, 64, 1000000, 13
NF = F + 1
DIN_PAD = 128

NC, NS = 2, 16
NW = NC * NS
TOTAL = B * F
PER_W = TOTAL // NW
CHUNK = 832
NCHUNK = PER_W // CHUNK


def _sc_gather_body(table_hbm, idx_hbm, out_hbm, idx_v, rows_v, sem):
  wid = lax.axis_index("s") * NC + lax.axis_index("c")
  base = wid * PER_W
  for ci in range(NCHUNK):
    off = base + ci * CHUNK
    pltpu.sync_copy(idx_hbm.at[pl.ds(off, CHUNK)], idx_v)
    pltpu.async_copy(table_hbm.at[idx_v], rows_v, sem).wait()
    pltpu.sync_copy(rows_v, out_hbm.at[pl.ds(off, CHUNK)])


def _sc_gather(table, idx_flat):
  mesh = plsc.VectorSubcoreMesh(core_axis_name="c", subcore_axis_name="s")
  fn = functools.partial(
      pl.kernel,
      mesh=mesh,
      out_type=jax.ShapeDtypeStruct((TOTAL, D), jnp.float32),
      scratch_types=[
          pltpu.VMEM((CHUNK,), jnp.int32),
          pltpu.VMEM((CHUNK, D), jnp.float32),
          pltpu.SemaphoreType.DMA,
      ],
      compiler_params=pltpu.CompilerParams(use_tc_tiling_on_sc=False),
  )(_sc_gather_body)
  return fn(table, idx_flat)


BT = 256
GRID = B // BT


def _tc_body(xd_ref, emb_ref, dw1_ref, db1_ref, dw2_ref, db2_ref, dw3_ref,
             db3_ref, ow1a_ref, w2_ref, ob1_ref, ow2_ref, ob2_ref, ow3_ref,
             ob3_ref, out_ref):
  f32 = jnp.float32
  x = xd_ref[...]
  h = jnp.maximum(jnp.dot(x, dw1_ref[...], preferred_element_type=f32)
                  + db1_ref[...], 0.0)
  h = jnp.maximum(jnp.dot(h, dw2_ref[...], preferred_element_type=f32)
                  + db2_ref[...], 0.0)
  do = jnp.maximum(jnp.dot(h, dw3_ref[...], preferred_element_type=f32)
                   + db3_ref[...], 0.0)
  emb = emb_ref[...]
  c3 = jnp.concatenate([do[:, None, :], emb.reshape(BT, F, D)], axis=1)
  acc = jnp.dot(do, ow1a_ref[...], preferred_element_type=f32)
  for n in range(NF):
    prod = c3 * c3[:, n:n + 1, :]
    gn = jnp.sum(prod, axis=-1)
    acc = acc + jnp.dot(gn, w2_ref[n], preferred_element_type=f32)
  h = jnp.maximum(acc + ob1_ref[...], 0.0)
  h = jnp.maximum(jnp.dot(h, ow2_ref[...], preferred_element_type=f32)
                  + ob2_ref[...], 0.0)
  out_ref[...] = (jnp.dot(h, ow3_ref[...], preferred_element_type=f32)
                  + ob3_ref[...])


def _tc_main(xd, emb2d, dw1p, db1, dw2, db2, dw3, db3, ow1a, w2, ob1, ow2,
             ob2, ow3p, ob3p, *, interpret=False):
  full = lambda shape: pl.BlockSpec(shape, lambda i: (0,) * len(shape))
  return pl.pallas_call(
      _tc_body,
      grid=(GRID,),
      in_specs=[
          pl.BlockSpec((BT, DIN_PAD), lambda i: (i, 0)),
          pl.BlockSpec((BT, F * D), lambda i: (i, 0)),
          full((DIN_PAD, 512)), full((1, 512)),
          full((512, 256)), full((1, 256)),
          full((256, D)), full((1, D)),
          full((D, 512)), full((NF, NF, 512)),
          full((1, 512)), full((512, 256)), full((1, 256)),
          full((256, 128)), full((1, 128)),
      ],
      out_specs=pl.BlockSpec((BT, 128), lambda i: (i, 0)),
      out_shape=jax.ShapeDtypeStruct((B, 128), jnp.float32),
      interpret=interpret,
  )(xd, emb2d, dw1p, db1, dw2, db2, dw3, db3, ow1a, w2, ob1, ow2, ob2,
    ow3p, ob3p)


def kernel(dense_features, sparse_indices, table, dw1, db1, dw2, db2, dw3,
           db3, ow1, ob1, ow2, ob2, ow3, ob3):
  idx_flat = sparse_indices.astype(jnp.int32).reshape(TOTAL)
  xd = jnp.pad(dense_features, ((0, 0), (0, DIN_PAD - DIN)))
  dw1p = jnp.pad(dw1, ((0, DIN_PAD - DIN), (0, 0)))
  ow1a = ow1[:D]
  iu0, iu1 = jnp.triu_indices(NF, k=1)
  w2 = jnp.zeros((NF * NF, 512), jnp.float32).at[iu0 * NF + iu1].set(ow1[D:])
  w2 = w2.reshape(NF, NF, 512)
  ow3p = jnp.pad(ow3, ((0, 0), (0, 127)))
  ob3p = jnp.pad(ob3, ((0, 127),)).reshape(1, 128)
  row = lambda b: b.reshape(1, -1)

  emb = _sc_gather(table, idx_flat)
  emb2d = emb.reshape(B, F * D)

  out = _tc_main(xd, emb2d, dw1p, row(db1), dw2, row(db2), dw3, row(db3),
                 ow1a, w2, row(ob1), ow2, row(ob2), ow3p, ob3p)
  return out[:, :1]


# speedup vs baseline: 1.5624x; 1.5624x over previous
import functools

import jax
import jax.numpy as jnp
import numpy as np
from jax import lax
from jax.experimental import pallas as pl
from jax.experimental.pallas import tpu as pltpu
from jax.experimental.pallas import tpu_sc as plsc

B, F, D, V, DIN = 4096, 26, 64, 1000000, 13
NF = F + 1
NFP = 32
NPAIR = NF * (NF - 1) // 2

_S = np.zeros((NPAIR, NFP * NF), np.float32)
_p = 0
for _n in range(NF):
  for _m in range(_n + 1, NF):
    _S[_p, _n * NFP + _m] = 1.0
    _p += 1

NC, NS = 2, 16
NW = NC * NS
TOTAL = B * F
PER_W = TOTAL // NW
CHUNK = 832
NCHUNK = PER_W // CHUNK


def _sc_gather_body(table_hbm, idx_hbm, out_hbm, idx_v, rows_v, sem):
  wid = lax.axis_index("s") * NC + lax.axis_index("c")
  base = wid * PER_W
  for ci in range(NCHUNK):
    off = base + ci * CHUNK
    pltpu.sync_copy(idx_hbm.at[pl.ds(off, CHUNK)], idx_v)
    pltpu.async_copy(table_hbm.at[idx_v], rows_v, sem).wait()
    pltpu.sync_copy(rows_v, out_hbm.at[pl.ds(off, CHUNK)])


def _sc_gather(table, idx_flat):
  mesh = plsc.VectorSubcoreMesh(core_axis_name="c", subcore_axis_name="s")
  fn = functools.partial(
      pl.kernel,
      mesh=mesh,
      out_type=jax.ShapeDtypeStruct((TOTAL, D), jnp.float32),
      scratch_types=[
          pltpu.VMEM((CHUNK,), jnp.int32),
          pltpu.VMEM((CHUNK, D), jnp.float32),
          pltpu.SemaphoreType.DMA,
      ],
      compiler_params=pltpu.CompilerParams(use_tc_tiling_on_sc=False),
  )(_sc_gather_body)
  return fn(table, idx_flat)


BT = 128
GRID = B // BT


def _tc_body(xdT_ref, emb_ref, dw1T_ref, db1_ref, dw2T_ref, db2_ref,
             dw3T_ref, db3_ref, ow1aT_ref, w2T_ref, ob1_ref, ow2T_ref,
             ob2_ref, ow3T_ref, ob3_ref, out_ref):
  f32 = jnp.float32
  dot = lambda a, b: jax.lax.dot_general(
      a, b, (((1,), (0,)), ((), ())), preferred_element_type=f32)
  h = jnp.maximum(dot(dw1T_ref[...], xdT_ref[...]) + db1_ref[...], 0.0)
  h = jnp.maximum(dot(dw2T_ref[...], h) + db2_ref[...], 0.0)
  doT = jnp.maximum(dot(dw3T_ref[...], h) + db3_ref[...], 0.0)
  embT = emb_ref[...].T
  cT = jnp.concatenate(
      [doT, embT, jnp.zeros(((NFP - NF) * D, BT), f32)], axis=0)
  c3 = cT.reshape(NFP, D, BT)
  gs = []
  for n in range(NF):
    prod = c3 * c3[n][None]
    gs.append(jnp.sum(prod, axis=1))
  g = jnp.concatenate(gs, axis=0)
  acc = dot(w2T_ref[...], g) + dot(ow1aT_ref[...], doT) + ob1_ref[...]
  h = jnp.maximum(acc, 0.0)
  h = jnp.maximum(dot(ow2T_ref[...], h) + ob2_ref[...], 0.0)
  out_ref[...] = dot(ow3T_ref[...], h) + ob3_ref[...]


def _tc_main(xdT, emb2d, dw1T, db1, dw2T, db2, dw3T, db3, ow1aT, w2T, ob1,
             ow2T, ob2, ow3T, ob3, *, interpret=False):
  full = lambda shape: pl.BlockSpec(shape, lambda i: (0,) * len(shape))
  return pl.pallas_call(
      _tc_body,
      grid=(GRID,),
      in_specs=[
          pl.BlockSpec((16, BT), lambda i: (0, i)),
          pl.BlockSpec((BT, F * D), lambda i: (i, 0)),
          full((512, 16)), full((512, 1)),
          full((256, 512)), full((256, 1)),
          full((D, 256)), full((D, 1)),
          full((512, D)), full((512, NF * NFP)),
          full((512, 1)), full((256, 512)), full((256, 1)),
          full((8, 256)), full((8, 1)),
      ],
      out_specs=pl.BlockSpec((8, BT), lambda i: (0, i)),
      out_shape=jax.ShapeDtypeStruct((8, B), jnp.float32),
      interpret=interpret,
  )(xdT, emb2d, dw1T, db1, dw2T, db2, dw3T, db3, ow1aT, w2T, ob1, ow2T,
    ob2, ow3T, ob3)


def kernel(dense_features, sparse_indices, table, dw1, db1, dw2, db2, dw3,
           db3, ow1, ob1, ow2, ob2, ow3, ob3):
  idx_flat = sparse_indices.astype(jnp.int32).reshape(TOTAL)
  xdT = jnp.pad(dense_features, ((0, 0), (0, 16 - DIN))).T
  dw1T = jnp.pad(dw1, ((0, 16 - DIN), (0, 0))).T
  ow1aT = ow1[:D].T
  w2T = jnp.dot(ow1[D:].T, jnp.asarray(_S))
  ow3T = jnp.pad(ow3, ((0, 0), (0, 7))).T
  ob3T = jnp.pad(ob3, ((0, 7),)).reshape(8, 1)
  col = lambda b: b.reshape(-1, 1)

  emb = _sc_gather(table, idx_flat)
  emb2d = emb.reshape(B, F * D)

  out = _tc_main(xdT, emb2d, dw1T.astype(jnp.float32), col(db1), dw2.T,
                 col(db2), dw3.T, col(db3), ow1aT, w2T, col(ob1), ow2.T,
                 col(ob2), ow3T, ob3T)
  return out[0].reshape(B, 1)
